# baseline (speedup 1.0000x reference)
import functools
from functools import partial

import jax
import jax.numpy as jnp
from jax import lax
from jax.experimental import pallas as pl
from jax.experimental.pallas import tpu as pltpu
from jax.experimental.pallas import tpu_sc as plsc

_SC_CORES = 2
_SC_SUBCORES = 16
_SC_WORKERS = _SC_CORES * _SC_SUBCORES
_GCHUNK = 80
_GROWS = 4000
_GROWS_PW = _GROWS // _SC_WORKERS


def _sc_gather_body(a_hbm, b_hbm, dix_hbm, six_hbm, ga_hbm, gb_hbm,
                    dix_v, six_v, bufa, bufb, gsem):
    wid = lax.axis_index("s") * _SC_CORES + lax.axis_index("c")
    row0 = wid * _GROWS_PW
    pltpu.sync_copy(dix_hbm.at[wid], dix_v)
    pltpu.sync_copy(six_hbm.at[wid], six_v)

    def body(k, _):
        c1 = pltpu.async_copy(a_hbm.at[dix_v.at[k]], bufa, gsem)
        c2 = pltpu.async_copy(b_hbm.at[six_v.at[k]], bufb, gsem)
        c1.wait()
        c2.wait()
        e0 = (row0 + k) * _GCHUNK
        pltpu.sync_copy(bufa, ga_hbm.at[pl.ds(e0, _GCHUNK)])
        pltpu.sync_copy(bufb, gb_hbm.at[pl.ds(e0, _GCHUNK)])
        return 0

    lax.fori_loop(0, _GROWS_PW, body, 0)


def _sc_gather(a, b, dix2d, six2d):
    e = _GROWS * _GCHUNK
    h = a.shape[1]
    mesh = plsc.VectorSubcoreMesh(core_axis_name="c", subcore_axis_name="s")
    fn = functools.partial(
        pl.kernel,
        mesh=mesh,
        compiler_params=pltpu.CompilerParams(use_tc_tiling_on_sc=False, needs_layout_passes=False),
        out_type=[
            jax.ShapeDtypeStruct((e, h), jnp.float32),
            jax.ShapeDtypeStruct((e, h), jnp.float32),
        ],
        scratch_types=[
            pltpu.VMEM((_GROWS_PW, _GCHUNK), jnp.int32),
            pltpu.VMEM((_GROWS_PW, _GCHUNK), jnp.int32),
            pltpu.VMEM((_GCHUNK, h), jnp.float32),
            pltpu.VMEM((_GCHUNK, h), jnp.float32),
            pltpu.SemaphoreType.DMA,
        ],
    )(_sc_gather_body)
    return fn(a, b, dix2d, six2d)

_BN_EPS = 1e-5
_N_NODES = 10000
_NODE_BLK = 1000
_EDGE_BLK = 2000


_NPW = 313
_SCAP = 16384
_MCHUNK = 80


def _zero_f32_2d(ref, nrows, ncols16):
    z = jnp.zeros((16,), jnp.float32)

    def body(r, _):
        for c in range(ncols16):
            ref[r, pl.ds(c * 16, 16)] = z
        return 0

    lax.fori_loop(0, nrows, body, 0)


def _apply_max(r3_hbm, eidbuf, dlbuf, tbl, gb, sem, cnt):
    nch = (cnt + _MCHUNK - 1) // _MCHUNK

    def fire(k, slot):
        return pltpu.async_copy(
            r3_hbm.at[eidbuf.at[pl.ds(k * _MCHUNK, _MCHUNK)]], gb.at[slot], sem)

    @pl.when(nch > 0)
    def _():
        fire(0, 0)

        def chunk(k, _):
            @pl.when(k + 1 < nch)
            def _():
                fire(k + 1, (k + 1) % 2)

            pltpu.make_async_copy(
                r3_hbm.at[eidbuf.at[pl.ds(k * _MCHUNK, _MCHUNK)]],
                gb.at[k % 2], sem).wait()
            rem = jnp.minimum(cnt - k * _MCHUNK, _MCHUNK)
            slot = k % 2

            def row(i, _):
                dl = dlbuf[pl.ds(k * _MCHUNK + i, 16)][0]
                for c in range(4):
                    sl = pl.ds(c * 16, 16)
                    tbl[dl, sl] = jnp.maximum(tbl[dl, sl], gb[slot, i, sl])
                return 0

            lax.fori_loop(0, rem, row, 0)
            return 0

        lax.fori_loop(0, nch, chunk, 0)


def _write_table(tbl, out_hbm, wid, lo):
    @pl.when(wid < _SC_WORKERS - 1)
    def _():
        pltpu.sync_copy(tbl.at[pl.ds(0, _NPW)], out_hbm.at[pl.ds(lo, _NPW)])

    @pl.when(wid == _SC_WORKERS - 1)
    def _():
        last = 10000 - (_SC_WORKERS - 1) * _NPW
        pltpu.sync_copy(tbl.at[pl.ds(0, last)], out_hbm.at[pl.ds(lo, last)])


def _sc_scatmax_scan_body(r3_hbm, dix_hbm, out_hbm, eidl_hbm, dll_hbm, cnt_hbm,
                          dstbuf, mbuf, eidbuf, dlbuf, cbuf, tbl, gb, sem):
    wid = lax.axis_index("s") * _SC_CORES + lax.axis_index("c")
    lo = wid * _NPW
    hi = jnp.minimum(lo + _NPW, 10000)
    _zero_f32_2d(tbl, _NPW + 7, 4)
    zi = jnp.zeros((16,), jnp.int32)

    def zb(i, _):
        eidbuf[pl.ds(i * 16, 16)] = zi
        return 0

    lax.fori_loop(0, _SCAP // 16, zb, 0)
    cbuf[pl.ds(0, 16)] = zi

    lane = lax.iota(jnp.int32, 16)

    def clamp01(v):
        return jnp.minimum(jnp.maximum(v, 0), 1)

    def slice_body(s, cl):
        pltpu.sync_copy(dix_hbm.at[s], dstbuf)

        def row_body(r, cl):
            for g in range(_GCHUNK // 16):
                d = dstbuf[r, pl.ds(g * 16, 16)]
                dl = d - lo
                m01 = clamp01(dl + 1) * clamp01(hi - d)
                inv = 1 - m01
                base = s * 10000 + r * _GCHUNK + g * 16
                packed = (dl * m01 + 319 * inv) * 524288 + (base + lane) * m01
                plsc.store_scatter(mbuf, [cl * 16 + lane], packed)
                cl = cl + m01
            return cl

        return lax.fori_loop(0, _GROWS_PW, row_body, cl)

    cl = lax.fori_loop(0, _SC_WORKERS, slice_body, jnp.zeros((16,), jnp.int32))

    cbuf[pl.ds(0, 16)] = cl
    clv = cbuf[pl.ds(0, 16)]
    maxrows = clv[0]
    for j in range(1, 16):
        maxrows = jnp.maximum(maxrows, clv[j])
    cnt = maxrows * 16

    def unp(g, _):
        pv = mbuf[pl.ds(g * 16, 16)]
        valid = clamp01(cl - g)
        eidbuf[pl.ds(g * 16, 16)] = (pv & 524287) * valid
        dlbuf[pl.ds(g * 16, 16)] = (pv >> 19) * valid + 319 * (1 - valid)
        return 0

    lax.fori_loop(0, maxrows, unp, 0)

    cbuf[pl.ds(0, 16)] = jnp.full((16,), cnt, jnp.int32)
    pltpu.sync_copy(eidbuf, eidl_hbm.at[wid])
    pltpu.sync_copy(dlbuf, dll_hbm.at[wid])
    pltpu.sync_copy(cbuf, cnt_hbm.at[wid])

    _apply_max(r3_hbm, eidbuf, dlbuf, tbl, gb, sem, cnt)
    _write_table(tbl, out_hbm, wid, lo)


def _sc_scatmax_reuse_body(r3_hbm, eidl_hbm, dll_hbm, cnt_hbm, out_hbm,
                           eidbuf, dlbuf, cbuf, tbl, gb, sem):
    wid = lax.axis_index("s") * _SC_CORES + lax.axis_index("c")
    lo = wid * _NPW
    _zero_f32_2d(tbl, _NPW + 7, 4)
    pltpu.sync_copy(eidl_hbm.at[wid], eidbuf)
    pltpu.sync_copy(dll_hbm.at[wid], dlbuf)
    pltpu.sync_copy(cnt_hbm.at[wid], cbuf)
    cnt = cbuf[pl.ds(0, 16)][0]
    _apply_max(r3_hbm, eidbuf, dlbuf, tbl, gb, sem, cnt)
    _write_table(tbl, out_hbm, wid, lo)


def _scat_scratch():
    return [
        pltpu.VMEM((_SCAP,), jnp.int32),
        pltpu.VMEM((_SCAP,), jnp.int32),
        pltpu.VMEM((16,), jnp.int32),
        pltpu.VMEM((_NPW + 7, 64), jnp.float32),
        pltpu.VMEM((2, _MCHUNK, 64), jnp.float32),
        pltpu.SemaphoreType.DMA,
    ]


def _sc_scatter_scan(r3, dix3d):
    mesh = plsc.VectorSubcoreMesh(core_axis_name="c", subcore_axis_name="s")
    fn = functools.partial(
        pl.kernel,
        mesh=mesh,
        compiler_params=pltpu.CompilerParams(use_tc_tiling_on_sc=False, needs_layout_passes=False),
        out_type=[
            jax.ShapeDtypeStruct((10000, 64), jnp.float32),
            jax.ShapeDtypeStruct((_SC_WORKERS, _SCAP), jnp.int32),
            jax.ShapeDtypeStruct((_SC_WORKERS, _SCAP), jnp.int32),
            jax.ShapeDtypeStruct((_SC_WORKERS, 16), jnp.int32),
        ],
        scratch_types=[pltpu.VMEM((_GROWS_PW, _GCHUNK), jnp.int32),
                       pltpu.VMEM((_SCAP,), jnp.int32)] + _scat_scratch(),
    )(_sc_scatmax_scan_body)
    return fn(r3, dix3d)


def _sc_scatter_reuse(r3, eidl, dll, cnts):
    mesh = plsc.VectorSubcoreMesh(core_axis_name="c", subcore_axis_name="s")
    fn = functools.partial(
        pl.kernel,
        mesh=mesh,
        compiler_params=pltpu.CompilerParams(use_tc_tiling_on_sc=False, needs_layout_passes=False),
        out_type=jax.ShapeDtypeStruct((10000, 64), jnp.float32),
        scratch_types=_scat_scratch(),
    )(_sc_scatmax_reuse_body)
    return fn(r3, eidl, dll, cnts)


def _precompute_body(pos_ref, feat_ref, wa_ref, wb_ref, wc_ref, b_ref, a_out, b_out):
    pos = pos_ref[...]
    feat = feat_ref[...]
    a = jnp.dot(pos, wa_ref[...], preferred_element_type=jnp.float32)
    a = a + jnp.dot(feat, wc_ref[...], preferred_element_type=jnp.float32)
    a_out[...] = a + b_ref[...]
    b_out[...] = jnp.dot(pos, wb_ref[...], preferred_element_type=jnp.float32)


def _node_precompute(pos, feat, w1, b1):
    n, f = feat.shape
    h = w1.shape[1]
    wa = w1[0:3] - w1[3:6]
    wb = w1[3:6]
    wc = w1[6:]
    grid = n // _NODE_BLK
    return pl.pallas_call(
        _precompute_body,
        grid=(grid,),
        in_specs=[
            pl.BlockSpec((_NODE_BLK, 3), lambda i: (i, 0)),
            pl.BlockSpec((_NODE_BLK, f), lambda i: (i, 0)),
            pl.BlockSpec((3, h), lambda i: (0, 0)),
            pl.BlockSpec((3, h), lambda i: (0, 0)),
            pl.BlockSpec((f, h), lambda i: (0, 0)),
            pl.BlockSpec((1, h), lambda i: (0, 0)),
        ],
        out_specs=[
            pl.BlockSpec((_NODE_BLK, h), lambda i: (i, 0)),
            pl.BlockSpec((_NODE_BLK, h), lambda i: (i, 0)),
        ],
        out_shape=[
            jax.ShapeDtypeStruct((n, h), jnp.float32),
            jax.ShapeDtypeStruct((n, h), jnp.float32),
        ],
    )(pos, feat, wa, wb, wc, b1.reshape(1, h))


def _edge_mlp_body(ga_ref, gb_ref, w2_ref, b2_ref, w3_ref, b3_ref, s3_ref, t3_ref, out_ref):
    h1 = jnp.maximum(ga_ref[...] + gb_ref[...], 0.0)
    u2 = jnp.dot(h1, w2_ref[...], preferred_element_type=jnp.float32) + b2_ref[...]
    h2 = jnp.maximum(u2, 0.0)
    u3 = jnp.dot(h2, w3_ref[...], preferred_element_type=jnp.float32) + b3_ref[...]
    h3 = s3_ref[...] * jnp.maximum(u3, 0.0) + t3_ref[...]
    out_ref[...] = jnp.maximum(h3, 0.0)


def _edge_mlp(ga, gb, w2f, b2f, w3f, b3f, s3, t3):
    e, h = ga.shape
    grid = e // _EDGE_BLK
    return pl.pallas_call(
        _edge_mlp_body,
        grid=(grid,),
        in_specs=[
            pl.BlockSpec((_EDGE_BLK, h), lambda i: (i, 0)),
            pl.BlockSpec((_EDGE_BLK, h), lambda i: (i, 0)),
            pl.BlockSpec((h, h), lambda i: (0, 0)),
            pl.BlockSpec((1, h), lambda i: (0, 0)),
            pl.BlockSpec((h, h), lambda i: (0, 0)),
            pl.BlockSpec((1, h), lambda i: (0, 0)),
            pl.BlockSpec((1, h), lambda i: (0, 0)),
            pl.BlockSpec((1, h), lambda i: (0, 0)),
        ],
        out_specs=pl.BlockSpec((_EDGE_BLK, h), lambda i: (i, 0)),
        out_shape=jax.ShapeDtypeStruct((e, h), jnp.float32),
    )(ga, gb, w2f, b2f.reshape(1, h), w3f, b3f.reshape(1, h),
      s3.reshape(1, h), t3.reshape(1, h))


def _head_body(m1_ref, m2_ref, lw1_ref, lb1_ref, lw2_ref, lb2_ref, out_ref):
    n = m1_ref.shape[0]
    mean1 = jnp.sum(m1_ref[...], axis=0, keepdims=True) * (1.0 / n)
    mean2 = jnp.sum(m2_ref[...], axis=0, keepdims=True) * (1.0 / n)
    j = jnp.concatenate([mean1, mean2], axis=1)
    z = jnp.maximum(jnp.dot(j, lw1_ref[...], preferred_element_type=jnp.float32)
                    + lb1_ref[...], 0.0)
    logits = jnp.dot(z, lw2_ref[...], preferred_element_type=jnp.float32) + lb2_ref[...]
    mx = jnp.max(logits, axis=1, keepdims=True)
    lse = jnp.log(jnp.sum(jnp.exp(logits - mx), axis=1, keepdims=True)) + mx
    out_ref[...] = logits - lse


def _head(m1, m2, lw1, lb1, lw2, lb2):
    n, h = m1.shape
    ncls = lw2.shape[1]
    return pl.pallas_call(
        _head_body,
        out_shape=jax.ShapeDtypeStruct((1, ncls), jnp.float32),
    )(m1, m2, lw1, lb1.reshape(1, h), lw2, lb2.reshape(1, ncls))


def _fold_bn(params):
    c = 1.0 / jnp.sqrt(1.0 + _BN_EPS)
    w1, b1, g1, be1 = params[0:4]
    w2, b2, g2, be2 = params[4:8]
    w3, b3, g3, be3 = params[8:12]
    s1, t1 = g1 * c, be1
    s2, t2 = g2 * c, be2
    s3, t3 = g3 * c, be3
    w2f = s1[:, None] * w2
    b2f = t1 @ w2 + b2
    w3f = s2[:, None] * w3
    b3f = t2 @ w3 + b3
    return w1, b1, w2f, b2f, w3f, b3f, s3, t3


def _conv_edge_mlp(pos, feat, dix2d, six2d, params):
    w1, b1, w2f, b2f, w3f, b3f, s3, t3 = _fold_bn(params)
    a, b = _node_precompute(pos, feat, w1, b1)
    ga, gb = _sc_gather(a, b, dix2d, six2d)
    return _edge_mlp(ga, gb, w2f, b2f, w3f, b3f, s3, t3)


def kernel(x, pos, edge_index, batch, p1, p2, lin):
    dst = edge_index[1]
    src = edge_index[0]
    dix2d = dst.reshape(_SC_WORKERS, _GROWS_PW, _GCHUNK)
    six2d = src.reshape(_SC_WORKERS, _GROWS_PW, _GCHUNK)
    r3a = _conv_edge_mlp(pos, x, dix2d, six2d, p1)
    h1, eidl, dll, cnts = _sc_scatter_scan(r3a, dix2d)
    r3b = _conv_edge_mlp(pos, h1, dix2d, six2d, p2)
    h2 = _sc_scatter_reuse(r3b, eidl, dll, cnts)
    lw1, lb1, lw2, lb2 = lin
    return _head(h1, h2, lw1, lb1, lw2, lb2)


# speedup vs baseline: 1.2669x; 1.2669x over previous
import functools
from functools import partial

import jax
import jax.numpy as jnp
from jax import lax
from jax.experimental import pallas as pl
from jax.experimental.pallas import tpu as pltpu
from jax.experimental.pallas import tpu_sc as plsc

_SC_CORES = 2
_SC_SUBCORES = 16
_SC_WORKERS = _SC_CORES * _SC_SUBCORES
_GCHUNK = 80
_GROWS = 4000
_GROWS_PW = _GROWS // _SC_WORKERS


def _sc_gather_body(a_hbm, b_hbm, dix_hbm, six_hbm, ga_hbm, gb_hbm,
                    dix_v, six_v, bufa, bufb, gsem):
    wid = lax.axis_index("s") * _SC_CORES + lax.axis_index("c")
    row0 = wid * _GROWS_PW
    pltpu.sync_copy(dix_hbm.at[wid], dix_v)
    pltpu.sync_copy(six_hbm.at[wid], six_v)

    def body(k, _):
        c1 = pltpu.async_copy(a_hbm.at[dix_v.at[k]], bufa, gsem)
        c2 = pltpu.async_copy(b_hbm.at[six_v.at[k]], bufb, gsem)
        c1.wait()
        c2.wait()
        e0 = (row0 + k) * _GCHUNK
        pltpu.sync_copy(bufa, ga_hbm.at[pl.ds(e0, _GCHUNK)])
        pltpu.sync_copy(bufb, gb_hbm.at[pl.ds(e0, _GCHUNK)])
        return 0

    lax.fori_loop(0, _GROWS_PW, body, 0)


def _sc_gather(a, b, dix2d, six2d):
    e = _GROWS * _GCHUNK
    h = a.shape[1]
    mesh = plsc.VectorSubcoreMesh(core_axis_name="c", subcore_axis_name="s")
    fn = functools.partial(
        pl.kernel,
        mesh=mesh,
        compiler_params=pltpu.CompilerParams(use_tc_tiling_on_sc=False, needs_layout_passes=False),
        out_type=[
            jax.ShapeDtypeStruct((e, h), jnp.float32),
            jax.ShapeDtypeStruct((e, h), jnp.float32),
        ],
        scratch_types=[
            pltpu.VMEM((_GROWS_PW, _GCHUNK), jnp.int32),
            pltpu.VMEM((_GROWS_PW, _GCHUNK), jnp.int32),
            pltpu.VMEM((_GCHUNK, h), jnp.float32),
            pltpu.VMEM((_GCHUNK, h), jnp.float32),
            pltpu.SemaphoreType.DMA,
        ],
    )(_sc_gather_body)
    return fn(a, b, dix2d, six2d)

_BN_EPS = 1e-5
_N_NODES = 10000
_NODE_BLK = 1000
_EDGE_BLK = 2000


_NPW = 313
_SCAP = 16384
_MCHUNK = 80


def _zero_f32_2d(ref, nrows, ncols16):
    z = jnp.zeros((16,), jnp.float32)

    def body(r, _):
        for c in range(ncols16):
            ref[r, pl.ds(c * 16, 16)] = z
        return 0

    lax.fori_loop(0, nrows, body, 0)


def _apply_max(r3_hbm, eidbuf, dlbuf, tbl, gb, sem, cnt):
    nch = (cnt + _MCHUNK - 1) // _MCHUNK

    def fire(k, slot):
        return pltpu.async_copy(
            r3_hbm.at[eidbuf.at[pl.ds(k * _MCHUNK, _MCHUNK)]], gb.at[slot], sem)

    @pl.when(nch > 0)
    def _():
        fire(0, 0)

        def chunk(k, _):
            @pl.when(k + 1 < nch)
            def _():
                fire(k + 1, (k + 1) % 2)

            pltpu.make_async_copy(
                r3_hbm.at[eidbuf.at[pl.ds(k * _MCHUNK, _MCHUNK)]],
                gb.at[k % 2], sem).wait()
            rem = jnp.minimum(cnt - k * _MCHUNK, _MCHUNK)
            slot = k % 2

            def row(i, _):
                dl = dlbuf[pl.ds(k * _MCHUNK + i, 16)][0]
                for c in range(4):
                    sl = pl.ds(c * 16, 16)
                    tbl[dl, sl] = jnp.maximum(tbl[dl, sl], gb[slot, i, sl])
                return 0

            lax.fori_loop(0, rem, row, 0)
            return 0

        lax.fori_loop(0, nch, chunk, 0)


def _write_table(tbl, out_hbm, wid, lo):
    @pl.when(wid < _SC_WORKERS - 1)
    def _():
        pltpu.sync_copy(tbl.at[pl.ds(0, _NPW)], out_hbm.at[pl.ds(lo, _NPW)])

    @pl.when(wid == _SC_WORKERS - 1)
    def _():
        last = 10000 - (_SC_WORKERS - 1) * _NPW
        pltpu.sync_copy(tbl.at[pl.ds(0, last)], out_hbm.at[pl.ds(lo, last)])


def _sc_scatmax_scan_body(r3_hbm, dix_hbm, out_hbm, eidl_hbm, dll_hbm, cnt_hbm,
                          dstbuf, mbuf, eidbuf, dlbuf, cbuf, tbl, gb, sem):
    wid = lax.axis_index("s") * _SC_CORES + lax.axis_index("c")
    lo = wid * _NPW
    hi = jnp.minimum(lo + _NPW, 10000)
    _zero_f32_2d(tbl, _NPW + 7, 4)
    zi = jnp.zeros((16,), jnp.int32)

    def zb(i, _):
        eidbuf[pl.ds(i * 16, 16)] = zi
        return 0

    lax.fori_loop(0, _SCAP // 16, zb, 0)
    cbuf[pl.ds(0, 16)] = zi

    lane = lax.iota(jnp.int32, 16)

    def slice_body(s, cntv):
        pltpu.sync_copy(dix_hbm.at[s], dstbuf)

        def row_body(r, cntv):
            for g in range(_GCHUNK // 16):
                d = dstbuf[r, pl.ds(g * 16, 16)]
                m = (d >= lo) & (d < hi)
                base = s * 10000 + r * _GCHUNK + g * 16
                pos = cntv + plsc.cumsum(m.astype(jnp.int32)) - 1
                plsc.store_scatter(eidbuf, [pos], base + lane, mask=m)
                plsc.store_scatter(dlbuf, [pos], d - lo, mask=m)
                cntv = cntv + plsc.all_reduce_population_count(m)
            return cntv

        return lax.fori_loop(0, _GROWS_PW, row_body, cntv)

    cntv = lax.fori_loop(0, _SC_WORKERS, slice_body,
                         jnp.zeros((16,), jnp.int32))

    cbuf[pl.ds(0, 16)] = cntv
    cnt = cbuf[pl.ds(0, 16)][0]

    pltpu.sync_copy(eidbuf, eidl_hbm.at[wid])
    pltpu.sync_copy(dlbuf, dll_hbm.at[wid])
    pltpu.sync_copy(cbuf, cnt_hbm.at[wid])

    _apply_max(r3_hbm, eidbuf, dlbuf, tbl, gb, sem, cnt)
    _write_table(tbl, out_hbm, wid, lo)


def _sc_scatmax_reuse_body(r3_hbm, eidl_hbm, dll_hbm, cnt_hbm, out_hbm,
                           eidbuf, dlbuf, cbuf, tbl, gb, sem):
    wid = lax.axis_index("s") * _SC_CORES + lax.axis_index("c")
    lo = wid * _NPW
    _zero_f32_2d(tbl, _NPW + 7, 4)
    pltpu.sync_copy(eidl_hbm.at[wid], eidbuf)
    pltpu.sync_copy(dll_hbm.at[wid], dlbuf)
    pltpu.sync_copy(cnt_hbm.at[wid], cbuf)
    cnt = cbuf[pl.ds(0, 16)][0]
    _apply_max(r3_hbm, eidbuf, dlbuf, tbl, gb, sem, cnt)
    _write_table(tbl, out_hbm, wid, lo)


def _scat_scratch():
    return [
        pltpu.VMEM((_SCAP,), jnp.int32),
        pltpu.VMEM((_SCAP,), jnp.int32),
        pltpu.VMEM((16,), jnp.int32),
        pltpu.VMEM((_NPW + 7, 64), jnp.float32),
        pltpu.VMEM((2, _MCHUNK, 64), jnp.float32),
        pltpu.SemaphoreType.DMA,
    ]


def _sc_scatter_scan(r3, dix3d):
    mesh = plsc.VectorSubcoreMesh(core_axis_name="c", subcore_axis_name="s")
    fn = functools.partial(
        pl.kernel,
        mesh=mesh,
        compiler_params=pltpu.CompilerParams(use_tc_tiling_on_sc=False, needs_layout_passes=False),
        out_type=[
            jax.ShapeDtypeStruct((10000, 64), jnp.float32),
            jax.ShapeDtypeStruct((_SC_WORKERS, _SCAP), jnp.int32),
            jax.ShapeDtypeStruct((_SC_WORKERS, _SCAP), jnp.int32),
            jax.ShapeDtypeStruct((_SC_WORKERS, 16), jnp.int32),
        ],
        scratch_types=[pltpu.VMEM((_GROWS_PW, _GCHUNK), jnp.int32),
                       pltpu.VMEM((_SCAP,), jnp.int32)] + _scat_scratch(),
    )(_sc_scatmax_scan_body)
    return fn(r3, dix3d)


def _sc_scatter_reuse(r3, eidl, dll, cnts):
    mesh = plsc.VectorSubcoreMesh(core_axis_name="c", subcore_axis_name="s")
    fn = functools.partial(
        pl.kernel,
        mesh=mesh,
        compiler_params=pltpu.CompilerParams(use_tc_tiling_on_sc=False, needs_layout_passes=False),
        out_type=jax.ShapeDtypeStruct((10000, 64), jnp.float32),
        scratch_types=_scat_scratch(),
    )(_sc_scatmax_reuse_body)
    return fn(r3, eidl, dll, cnts)


def _precompute_body(pos_ref, feat_ref, wa_ref, wb_ref, wc_ref, b_ref, a_out, b_out):
    pos = pos_ref[...]
    feat = feat_ref[...]
    a = jnp.dot(pos, wa_ref[...], preferred_element_type=jnp.float32)
    a = a + jnp.dot(feat, wc_ref[...], preferred_element_type=jnp.float32)
    a_out[...] = a + b_ref[...]
    b_out[...] = jnp.dot(pos, wb_ref[...], preferred_element_type=jnp.float32)


def _node_precompute(pos, feat, w1, b1):
    n, f = feat.shape
    h = w1.shape[1]
    wa = w1[0:3] - w1[3:6]
    wb = w1[3:6]
    wc = w1[6:]
    grid = n // _NODE_BLK
    return pl.pallas_call(
        _precompute_body,
        grid=(grid,),
        in_specs=[
            pl.BlockSpec((_NODE_BLK, 3), lambda i: (i, 0)),
            pl.BlockSpec((_NODE_BLK, f), lambda i: (i, 0)),
            pl.BlockSpec((3, h), lambda i: (0, 0)),
            pl.BlockSpec((3, h), lambda i: (0, 0)),
            pl.BlockSpec((f, h), lambda i: (0, 0)),
            pl.BlockSpec((1, h), lambda i: (0, 0)),
        ],
        out_specs=[
            pl.BlockSpec((_NODE_BLK, h), lambda i: (i, 0)),
            pl.BlockSpec((_NODE_BLK, h), lambda i: (i, 0)),
        ],
        out_shape=[
            jax.ShapeDtypeStruct((n, h), jnp.float32),
            jax.ShapeDtypeStruct((n, h), jnp.float32),
        ],
    )(pos, feat, wa, wb, wc, b1.reshape(1, h))


def _edge_mlp_body(ga_ref, gb_ref, w2_ref, b2_ref, w3_ref, b3_ref, s3_ref, t3_ref, out_ref):
    h1 = jnp.maximum(ga_ref[...] + gb_ref[...], 0.0)
    u2 = jnp.dot(h1, w2_ref[...], preferred_element_type=jnp.float32) + b2_ref[...]
    h2 = jnp.maximum(u2, 0.0)
    u3 = jnp.dot(h2, w3_ref[...], preferred_element_type=jnp.float32) + b3_ref[...]
    h3 = s3_ref[...] * jnp.maximum(u3, 0.0) + t3_ref[...]
    out_ref[...] = jnp.maximum(h3, 0.0)


def _edge_mlp(ga, gb, w2f, b2f, w3f, b3f, s3, t3):
    e, h = ga.shape
    grid = e // _EDGE_BLK
    return pl.pallas_call(
        _edge_mlp_body,
        grid=(grid,),
        in_specs=[
            pl.BlockSpec((_EDGE_BLK, h), lambda i: (i, 0)),
            pl.BlockSpec((_EDGE_BLK, h), lambda i: (i, 0)),
            pl.BlockSpec((h, h), lambda i: (0, 0)),
            pl.BlockSpec((1, h), lambda i: (0, 0)),
            pl.BlockSpec((h, h), lambda i: (0, 0)),
            pl.BlockSpec((1, h), lambda i: (0, 0)),
            pl.BlockSpec((1, h), lambda i: (0, 0)),
            pl.BlockSpec((1, h), lambda i: (0, 0)),
        ],
        out_specs=pl.BlockSpec((_EDGE_BLK, h), lambda i: (i, 0)),
        out_shape=jax.ShapeDtypeStruct((e, h), jnp.float32),
    )(ga, gb, w2f, b2f.reshape(1, h), w3f, b3f.reshape(1, h),
      s3.reshape(1, h), t3.reshape(1, h))


def _head_body(m1_ref, m2_ref, lw1_ref, lb1_ref, lw2_ref, lb2_ref, out_ref):
    n = m1_ref.shape[0]
    mean1 = jnp.sum(m1_ref[...], axis=0, keepdims=True) * (1.0 / n)
    mean2 = jnp.sum(m2_ref[...], axis=0, keepdims=True) * (1.0 / n)
    j = jnp.concatenate([mean1, mean2], axis=1)
    z = jnp.maximum(jnp.dot(j, lw1_ref[...], preferred_element_type=jnp.float32)
                    + lb1_ref[...], 0.0)
    logits = jnp.dot(z, lw2_ref[...], preferred_element_type=jnp.float32) + lb2_ref[...]
    mx = jnp.max(logits, axis=1, keepdims=True)
    lse = jnp.log(jnp.sum(jnp.exp(logits - mx), axis=1, keepdims=True)) + mx
    out_ref[...] = logits - lse


def _head(m1, m2, lw1, lb1, lw2, lb2):
    n, h = m1.shape
    ncls = lw2.shape[1]
    return pl.pallas_call(
        _head_body,
        out_shape=jax.ShapeDtypeStruct((1, ncls), jnp.float32),
    )(m1, m2, lw1, lb1.reshape(1, h), lw2, lb2.reshape(1, ncls))


def _fold_bn(params):
    c = 1.0 / jnp.sqrt(1.0 + _BN_EPS)
    w1, b1, g1, be1 = params[0:4]
    w2, b2, g2, be2 = params[4:8]
    w3, b3, g3, be3 = params[8:12]
    s1, t1 = g1 * c, be1
    s2, t2 = g2 * c, be2
    s3, t3 = g3 * c, be3
    w2f = s1[:, None] * w2
    b2f = t1 @ w2 + b2
    w3f = s2[:, None] * w3
    b3f = t2 @ w3 + b3
    return w1, b1, w2f, b2f, w3f, b3f, s3, t3


def _conv_edge_mlp(pos, feat, dix2d, six2d, params):
    w1, b1, w2f, b2f, w3f, b3f, s3, t3 = _fold_bn(params)
    a, b = _node_precompute(pos, feat, w1, b1)
    ga, gb = _sc_gather(a, b, dix2d, six2d)
    return _edge_mlp(ga, gb, w2f, b2f, w3f, b3f, s3, t3)


def kernel(x, pos, edge_index, batch, p1, p2, lin):
    dst = edge_index[1]
    src = edge_index[0]
    dix2d = dst.reshape(_SC_WORKERS, _GROWS_PW, _GCHUNK)
    six2d = src.reshape(_SC_WORKERS, _GROWS_PW, _GCHUNK)
    r3a = _conv_edge_mlp(pos, x, dix2d, six2d, p1)
    h1, eidl, dll, cnts = _sc_scatter_scan(r3a, dix2d)
    r3b = _conv_edge_mlp(pos, h1, dix2d, six2d, p2)
    h2 = _sc_scatter_reuse(r3b, eidl, dll, cnts)
    lw1, lb1, lw2, lb2 = lin
    return _head(h1, h2, lw1, lb1, lw2, lb2)


# speedup vs baseline: 1.5080x; 1.1903x over previous
import functools
from functools import partial

import jax
import jax.numpy as jnp
from jax import lax
from jax.experimental import pallas as pl
from jax.experimental.pallas import tpu as pltpu
from jax.experimental.pallas import tpu_sc as plsc

_SC_CORES = 2
_SC_SUBCORES = 16
_SC_WORKERS = _SC_CORES * _SC_SUBCORES
_GCHUNK = 80
_GROWS = 4000
_GROWS_PW = _GROWS // _SC_WORKERS


def _sc_gather_body(a_hbm, b_hbm, dix_hbm, six_hbm, ga_hbm, gb_hbm,
                    dix_v, six_v, bufa, bufb, gsem, osem):
    wid = lax.axis_index("s") * _SC_CORES + lax.axis_index("c")
    row0 = wid * _GROWS_PW
    pltpu.sync_copy(dix_hbm.at[wid], dix_v)
    pltpu.sync_copy(six_hbm.at[wid], six_v)
    n = _GROWS_PW
    nb = 3

    def fire(k):
        slot = k % nb
        pltpu.async_copy(a_hbm.at[dix_v.at[k]], bufa.at[slot], gsem)
        pltpu.async_copy(b_hbm.at[six_v.at[k]], bufb.at[slot], gsem)

    def wait_in(k):
        slot = k % nb
        pltpu.make_async_copy(a_hbm.at[dix_v.at[k]], bufa.at[slot], gsem).wait()
        pltpu.make_async_copy(b_hbm.at[six_v.at[k]], bufb.at[slot], gsem).wait()

    def out_desc(k):
        slot = k % nb
        e0 = (row0 + k) * _GCHUNK
        da = (bufa.at[slot], ga_hbm.at[pl.ds(e0, _GCHUNK)])
        db = (bufb.at[slot], gb_hbm.at[pl.ds(e0, _GCHUNK)])
        return da, db

    fire(0)
    fire(1)

    def body(k, _):
        wait_in(k)
        (sa, dsta), (sb, dstb) = out_desc(k)
        pltpu.async_copy(sa, dsta, osem)
        pltpu.async_copy(sb, dstb, osem)

        @pl.when(k >= 1)
        def _():
            (pa, pda), (pb, pdb) = out_desc(k - 1)
            pltpu.make_async_copy(pa, pda, osem).wait()
            pltpu.make_async_copy(pb, pdb, osem).wait()

        @pl.when(k + 2 < n)
        def _():
            fire(k + 2)

        return 0

    lax.fori_loop(0, n, body, 0)
    (la, lda), (lb, ldb) = out_desc(n - 1)
    pltpu.make_async_copy(la, lda, osem).wait()
    pltpu.make_async_copy(lb, ldb, osem).wait()


def _sc_gather(a, b, dix2d, six2d):
    e = _GROWS * _GCHUNK
    h = a.shape[1]
    mesh = plsc.VectorSubcoreMesh(core_axis_name="c", subcore_axis_name="s")
    fn = functools.partial(
        pl.kernel,
        mesh=mesh,
        compiler_params=pltpu.CompilerParams(use_tc_tiling_on_sc=False, needs_layout_passes=False),
        out_type=[
            jax.ShapeDtypeStruct((e, h), jnp.float32),
            jax.ShapeDtypeStruct((e, h), jnp.float32),
        ],
        scratch_types=[
            pltpu.VMEM((_GROWS_PW, _GCHUNK), jnp.int32),
            pltpu.VMEM((_GROWS_PW, _GCHUNK), jnp.int32),
            pltpu.VMEM((3, _GCHUNK, h), jnp.float32),
            pltpu.VMEM((3, _GCHUNK, h), jnp.float32),
            pltpu.SemaphoreType.DMA,
            pltpu.SemaphoreType.DMA,
        ],
    )(_sc_gather_body)
    return fn(a, b, dix2d, six2d)

_BN_EPS = 1e-5
_N_NODES = 10000
_NODE_BLK = 1000
_EDGE_BLK = 2000


_NPW = 313
_SCAP = 16384
_MCHUNK = 80


def _zero_f32_2d(ref, nrows, ncols16):
    z = jnp.zeros((16,), jnp.float32)

    def body(r, _):
        for c in range(ncols16):
            ref[r, pl.ds(c * 16, 16)] = z
        return 0

    lax.fori_loop(0, nrows, body, 0)


def _apply_max(r3_hbm, eidbuf, dlbuf, tbl, gb, sem, cnt):
    nch = (cnt + _MCHUNK - 1) // _MCHUNK

    def fire(k, slot):
        return pltpu.async_copy(
            r3_hbm.at[eidbuf.at[pl.ds(k * _MCHUNK, _MCHUNK)]], gb.at[slot], sem)

    @pl.when(nch > 0)
    def _():
        fire(0, 0)

        def chunk(k, _):
            @pl.when(k + 1 < nch)
            def _():
                fire(k + 1, (k + 1) % 2)

            pltpu.make_async_copy(
                r3_hbm.at[eidbuf.at[pl.ds(k * _MCHUNK, _MCHUNK)]],
                gb.at[k % 2], sem).wait()
            slot = k % 2
            for b in range(_MCHUNK // 16):
                dls = dlbuf[pl.ds(k * _MCHUNK + b * 16, 16)]
                for j in range(16):
                    dl = dls[j]
                    i = b * 16 + j
                    for c in range(4):
                        sl = pl.ds(c * 16, 16)
                        tbl[dl, sl] = jnp.maximum(tbl[dl, sl], gb[slot, i, sl])
            return 0

        lax.fori_loop(0, nch, chunk, 0)


def _write_table(tbl, out_hbm, wid, lo):
    @pl.when(wid < _SC_WORKERS - 1)
    def _():
        pltpu.sync_copy(tbl.at[pl.ds(0, _NPW)], out_hbm.at[pl.ds(lo, _NPW)])

    @pl.when(wid == _SC_WORKERS - 1)
    def _():
        last = 10000 - (_SC_WORKERS - 1) * _NPW
        pltpu.sync_copy(tbl.at[pl.ds(0, last)], out_hbm.at[pl.ds(lo, last)])


def _sc_scatmax_scan_body(r3_hbm, dix_hbm, out_hbm, eidl_hbm, dll_hbm, cnt_hbm,
                          dstbuf, mbuf, eidbuf, dlbuf, cbuf, tbl, gb, sem):
    wid = lax.axis_index("s") * _SC_CORES + lax.axis_index("c")
    lo = wid * _NPW
    hi = jnp.minimum(lo + _NPW, 10000)
    _zero_f32_2d(tbl, _NPW + 7, 4)
    zi = jnp.zeros((16,), jnp.int32)

    s319 = jnp.full((16,), 319, jnp.int32)

    def zb(i, _):
        eidbuf[pl.ds(i * 16, 16)] = zi
        dlbuf[pl.ds(i * 16, 16)] = s319
        return 0

    lax.fori_loop(0, _SCAP // 16, zb, 0)
    cbuf[pl.ds(0, 16)] = zi

    lane = lax.iota(jnp.int32, 16)

    def slice_body(s, cntv):
        pltpu.sync_copy(dix_hbm.at[s], dstbuf)

        def row_body(r, cntv):
            for g in range(_GCHUNK // 16):
                d = dstbuf[r, pl.ds(g * 16, 16)]
                m = (d >= lo) & (d < hi)
                base = s * 10000 + r * _GCHUNK + g * 16
                pos = cntv + plsc.cumsum(m.astype(jnp.int32)) - 1
                plsc.store_scatter(eidbuf, [pos], base + lane, mask=m)
                plsc.store_scatter(dlbuf, [pos], d - lo, mask=m)
                cntv = cntv + plsc.all_reduce_population_count(m)
            return cntv

        return lax.fori_loop(0, _GROWS_PW, row_body, cntv)

    cntv = lax.fori_loop(0, _SC_WORKERS, slice_body,
                         jnp.zeros((16,), jnp.int32))

    cbuf[pl.ds(0, 16)] = cntv
    cnt = cbuf[pl.ds(0, 16)][0]

    pltpu.sync_copy(eidbuf, eidl_hbm.at[wid])
    pltpu.sync_copy(dlbuf, dll_hbm.at[wid])
    pltpu.sync_copy(cbuf, cnt_hbm.at[wid])

    _apply_max(r3_hbm, eidbuf, dlbuf, tbl, gb, sem, cnt)
    _write_table(tbl, out_hbm, wid, lo)


def _sc_scatmax_reuse_body(r3_hbm, eidl_hbm, dll_hbm, cnt_hbm, out_hbm,
                           eidbuf, dlbuf, cbuf, tbl, gb, sem):
    wid = lax.axis_index("s") * _SC_CORES + lax.axis_index("c")
    lo = wid * _NPW
    _zero_f32_2d(tbl, _NPW + 7, 4)
    pltpu.sync_copy(eidl_hbm.at[wid], eidbuf)
    pltpu.sync_copy(dll_hbm.at[wid], dlbuf)
    pltpu.sync_copy(cnt_hbm.at[wid], cbuf)
    cnt = cbuf[pl.ds(0, 16)][0]
    _apply_max(r3_hbm, eidbuf, dlbuf, tbl, gb, sem, cnt)
    _write_table(tbl, out_hbm, wid, lo)


def _scat_scratch():
    return [
        pltpu.VMEM((_SCAP,), jnp.int32),
        pltpu.VMEM((_SCAP,), jnp.int32),
        pltpu.VMEM((16,), jnp.int32),
        pltpu.VMEM((_NPW + 7, 64), jnp.float32),
        pltpu.VMEM((2, _MCHUNK, 64), jnp.float32),
        pltpu.SemaphoreType.DMA,
    ]


def _sc_scatter_scan(r3, dix3d):
    mesh = plsc.VectorSubcoreMesh(core_axis_name="c", subcore_axis_name="s")
    fn = functools.partial(
        pl.kernel,
        mesh=mesh,
        compiler_params=pltpu.CompilerParams(use_tc_tiling_on_sc=False, needs_layout_passes=False),
        out_type=[
            jax.ShapeDtypeStruct((10000, 64), jnp.float32),
            jax.ShapeDtypeStruct((_SC_WORKERS, _SCAP), jnp.int32),
            jax.ShapeDtypeStruct((_SC_WORKERS, _SCAP), jnp.int32),
            jax.ShapeDtypeStruct((_SC_WORKERS, 16), jnp.int32),
        ],
        scratch_types=[pltpu.VMEM((_GROWS_PW, _GCHUNK), jnp.int32),
                       pltpu.VMEM((_SCAP,), jnp.int32)] + _scat_scratch(),
    )(_sc_scatmax_scan_body)
    return fn(r3, dix3d)


def _sc_scatter_reuse(r3, eidl, dll, cnts):
    mesh = plsc.VectorSubcoreMesh(core_axis_name="c", subcore_axis_name="s")
    fn = functools.partial(
        pl.kernel,
        mesh=mesh,
        compiler_params=pltpu.CompilerParams(use_tc_tiling_on_sc=False, needs_layout_passes=False),
        out_type=jax.ShapeDtypeStruct((10000, 64), jnp.float32),
        scratch_types=_scat_scratch(),
    )(_sc_scatmax_reuse_body)
    return fn(r3, eidl, dll, cnts)


def _precompute_body(pos_ref, feat_ref, wa_ref, wb_ref, wc_ref, b_ref, a_out, b_out):
    pos = pos_ref[...]
    feat = feat_ref[...]
    a = jnp.dot(pos, wa_ref[...], preferred_element_type=jnp.float32)
    a = a + jnp.dot(feat, wc_ref[...], preferred_element_type=jnp.float32)
    a_out[...] = a + b_ref[...]
    b_out[...] = jnp.dot(pos, wb_ref[...], preferred_element_type=jnp.float32)


def _node_precompute(pos, feat, w1, b1):
    n, f = feat.shape
    h = w1.shape[1]
    wa = w1[0:3] - w1[3:6]
    wb = w1[3:6]
    wc = w1[6:]
    grid = n // _NODE_BLK
    return pl.pallas_call(
        _precompute_body,
        grid=(grid,),
        in_specs=[
            pl.BlockSpec((_NODE_BLK, 3), lambda i: (i, 0)),
            pl.BlockSpec((_NODE_BLK, f), lambda i: (i, 0)),
            pl.BlockSpec((3, h), lambda i: (0, 0)),
            pl.BlockSpec((3, h), lambda i: (0, 0)),
            pl.BlockSpec((f, h), lambda i: (0, 0)),
            pl.BlockSpec((1, h), lambda i: (0, 0)),
        ],
        out_specs=[
            pl.BlockSpec((_NODE_BLK, h), lambda i: (i, 0)),
            pl.BlockSpec((_NODE_BLK, h), lambda i: (i, 0)),
        ],
        out_shape=[
            jax.ShapeDtypeStruct((n, h), jnp.float32),
            jax.ShapeDtypeStruct((n, h), jnp.float32),
        ],
    )(pos, feat, wa, wb, wc, b1.reshape(1, h))


def _edge_mlp_body(ga_ref, gb_ref, w2_ref, b2_ref, w3_ref, b3_ref, s3_ref, t3_ref, out_ref):
    h1 = jnp.maximum(ga_ref[...] + gb_ref[...], 0.0)
    u2 = jnp.dot(h1, w2_ref[...], preferred_element_type=jnp.float32) + b2_ref[...]
    h2 = jnp.maximum(u2, 0.0)
    u3 = jnp.dot(h2, w3_ref[...], preferred_element_type=jnp.float32) + b3_ref[...]
    h3 = s3_ref[...] * jnp.maximum(u3, 0.0) + t3_ref[...]
    out_ref[...] = jnp.maximum(h3, 0.0)


def _edge_mlp(ga, gb, w2f, b2f, w3f, b3f, s3, t3):
    e, h = ga.shape
    grid = e // _EDGE_BLK
    return pl.pallas_call(
        _edge_mlp_body,
        grid=(grid,),
        in_specs=[
            pl.BlockSpec((_EDGE_BLK, h), lambda i: (i, 0)),
            pl.BlockSpec((_EDGE_BLK, h), lambda i: (i, 0)),
            pl.BlockSpec((h, h), lambda i: (0, 0)),
            pl.BlockSpec((1, h), lambda i: (0, 0)),
            pl.BlockSpec((h, h), lambda i: (0, 0)),
            pl.BlockSpec((1, h), lambda i: (0, 0)),
            pl.BlockSpec((1, h), lambda i: (0, 0)),
            pl.BlockSpec((1, h), lambda i: (0, 0)),
        ],
        out_specs=pl.BlockSpec((_EDGE_BLK, h), lambda i: (i, 0)),
        out_shape=jax.ShapeDtypeStruct((e, h), jnp.float32),
    )(ga, gb, w2f, b2f.reshape(1, h), w3f, b3f.reshape(1, h),
      s3.reshape(1, h), t3.reshape(1, h))


def _head_body(m1_ref, m2_ref, lw1_ref, lb1_ref, lw2_ref, lb2_ref, out_ref):
    n = m1_ref.shape[0]
    mean1 = jnp.sum(m1_ref[...], axis=0, keepdims=True) * (1.0 / n)
    mean2 = jnp.sum(m2_ref[...], axis=0, keepdims=True) * (1.0 / n)
    j = jnp.concatenate([mean1, mean2], axis=1)
    z = jnp.maximum(jnp.dot(j, lw1_ref[...], preferred_element_type=jnp.float32)
                    + lb1_ref[...], 0.0)
    logits = jnp.dot(z, lw2_ref[...], preferred_element_type=jnp.float32) + lb2_ref[...]
    mx = jnp.max(logits, axis=1, keepdims=True)
    lse = jnp.log(jnp.sum(jnp.exp(logits - mx), axis=1, keepdims=True)) + mx
    out_ref[...] = logits - lse


def _head(m1, m2, lw1, lb1, lw2, lb2):
    n, h = m1.shape
    ncls = lw2.shape[1]
    return pl.pallas_call(
        _head_body,
        out_shape=jax.ShapeDtypeStruct((1, ncls), jnp.float32),
    )(m1, m2, lw1, lb1.reshape(1, h), lw2, lb2.reshape(1, ncls))


def _fold_bn(params):
    c = 1.0 / jnp.sqrt(1.0 + _BN_EPS)
    w1, b1, g1, be1 = params[0:4]
    w2, b2, g2, be2 = params[4:8]
    w3, b3, g3, be3 = params[8:12]
    s1, t1 = g1 * c, be1
    s2, t2 = g2 * c, be2
    s3, t3 = g3 * c, be3
    w2f = s1[:, None] * w2
    b2f = t1 @ w2 + b2
    w3f = s2[:, None] * w3
    b3f = t2 @ w3 + b3
    return w1, b1, w2f, b2f, w3f, b3f, s3, t3


def _conv_edge_mlp(pos, feat, dix2d, six2d, params):
    w1, b1, w2f, b2f, w3f, b3f, s3, t3 = _fold_bn(params)
    a, b = _node_precompute(pos, feat, w1, b1)
    ga, gb = _sc_gather(a, b, dix2d, six2d)
    return _edge_mlp(ga, gb, w2f, b2f, w3f, b3f, s3, t3)


def kernel(x, pos, edge_index, batch, p1, p2, lin):
    dst = edge_index[1]
    src = edge_index[0]
    dix2d = dst.reshape(_SC_WORKERS, _GROWS_PW, _GCHUNK)
    six2d = src.reshape(_SC_WORKERS, _GROWS_PW, _GCHUNK)
    r3a = _conv_edge_mlp(pos, x, dix2d, six2d, p1)
    h1, eidl, dll, cnts = _sc_scatter_scan(r3a, dix2d)
    r3b = _conv_edge_mlp(pos, h1, dix2d, six2d, p2)
    h2 = _sc_scatter_reuse(r3b, eidl, dll, cnts)
    lw1, lb1, lw2, lb2 = lin
    return _head(h1, h2, lw1, lb1, lw2, lb2)


# speedup vs baseline: 1.5096x; 1.0010x over previous
import functools
from functools import partial

import jax
import jax.numpy as jnp
from jax import lax
from jax.experimental import pallas as pl
from jax.experimental.pallas import tpu as pltpu
from jax.experimental.pallas import tpu_sc as plsc

_SC_CORES = 2
_SC_SUBCORES = 16
_SC_WORKERS = _SC_CORES * _SC_SUBCORES
_GCHUNK = 80
_GROWS = 4000
_GROWS_PW = _GROWS // _SC_WORKERS


def _sc_gather_body(a_hbm, b_hbm, dix_hbm, six_hbm, ga_hbm, gb_hbm,
                    dix_v, six_v, bufa, bufb, gsem, osem):
    wid = lax.axis_index("s") * _SC_CORES + lax.axis_index("c")
    row0 = wid * _GROWS_PW
    pltpu.sync_copy(dix_hbm.at[wid], dix_v)
    pltpu.sync_copy(six_hbm.at[wid], six_v)
    n = _GROWS_PW
    nb = 3

    def fire(k):
        slot = k % nb
        pltpu.async_copy(a_hbm.at[dix_v.at[k]], bufa.at[slot], gsem)
        pltpu.async_copy(b_hbm.at[six_v.at[k]], bufb.at[slot], gsem)

    def wait_in(k):
        slot = k % nb
        pltpu.make_async_copy(a_hbm.at[dix_v.at[k]], bufa.at[slot], gsem).wait()
        pltpu.make_async_copy(b_hbm.at[six_v.at[k]], bufb.at[slot], gsem).wait()

    def out_desc(k):
        slot = k % nb
        e0 = (row0 + k) * _GCHUNK
        da = (bufa.at[slot], ga_hbm.at[pl.ds(e0, _GCHUNK)])
        db = (bufb.at[slot], gb_hbm.at[pl.ds(e0, _GCHUNK)])
        return da, db

    fire(0)
    fire(1)

    def body(k, _):
        wait_in(k)
        (sa, dsta), (sb, dstb) = out_desc(k)
        pltpu.async_copy(sa, dsta, osem)
        pltpu.async_copy(sb, dstb, osem)

        @pl.when(k >= 1)
        def _():
            (pa, pda), (pb, pdb) = out_desc(k - 1)
            pltpu.make_async_copy(pa, pda, osem).wait()
            pltpu.make_async_copy(pb, pdb, osem).wait()

        @pl.when(k + 2 < n)
        def _():
            fire(k + 2)

        return 0

    lax.fori_loop(0, n, body, 0)
    (la, lda), (lb, ldb) = out_desc(n - 1)
    pltpu.make_async_copy(la, lda, osem).wait()
    pltpu.make_async_copy(lb, ldb, osem).wait()


def _sc_gather(a, b, dix2d, six2d):
    e = _GROWS * _GCHUNK
    h = a.shape[1]
    mesh = plsc.VectorSubcoreMesh(core_axis_name="c", subcore_axis_name="s")
    fn = functools.partial(
        pl.kernel,
        mesh=mesh,
        compiler_params=pltpu.CompilerParams(use_tc_tiling_on_sc=False, needs_layout_passes=False),
        out_type=[
            jax.ShapeDtypeStruct((e, h), jnp.float32),
            jax.ShapeDtypeStruct((e, h), jnp.float32),
        ],
        scratch_types=[
            pltpu.VMEM((_GROWS_PW, _GCHUNK), jnp.int32),
            pltpu.VMEM((_GROWS_PW, _GCHUNK), jnp.int32),
            pltpu.VMEM((3, _GCHUNK, h), jnp.float32),
            pltpu.VMEM((3, _GCHUNK, h), jnp.float32),
            pltpu.SemaphoreType.DMA,
            pltpu.SemaphoreType.DMA,
        ],
    )(_sc_gather_body)
    return fn(a, b, dix2d, six2d)

_BN_EPS = 1e-5
_N_NODES = 10000
_NODE_BLK = 1000
_EDGE_BLK = 2000


_NPW = 313
_SCAP = 16384
_MCHUNK = 80


def _zero_f32_2d(ref, nrows, ncols16):
    z = jnp.zeros((16,), jnp.float32)

    def body(r, _):
        for c in range(ncols16):
            ref[r, pl.ds(c * 16, 16)] = z
        return 0

    lax.fori_loop(0, nrows, body, 0)


def _apply_max(r3_hbm, eidbuf, dlbuf, tbl, gb, sem, cnt):
    nch = (cnt + _MCHUNK - 1) // _MCHUNK

    def fire(k, slot):
        return pltpu.async_copy(
            r3_hbm.at[eidbuf.at[pl.ds(k * _MCHUNK, _MCHUNK)]], gb.at[slot], sem)

    @pl.when(nch > 0)
    def _():
        fire(0, 0)

        def chunk(k, _):
            @pl.when(k + 1 < nch)
            def _():
                fire(k + 1, (k + 1) % 2)

            pltpu.make_async_copy(
                r3_hbm.at[eidbuf.at[pl.ds(k * _MCHUNK, _MCHUNK)]],
                gb.at[k % 2], sem).wait()
            slot = k % 2
            for b in range(_MCHUNK // 16):
                dls = dlbuf[pl.ds(k * _MCHUNK + b * 16, 16)]
                for j in range(16):
                    dl = dls[j]
                    i = b * 16 + j
                    for c in range(4):
                        sl = pl.ds(c * 16, 16)
                        tbl[dl, sl] = jnp.maximum(tbl[dl, sl], gb[slot, i, sl])
            return 0

        lax.fori_loop(0, nch, chunk, 0)


def _write_table(tbl, out_hbm, wid, lo):
    @pl.when(wid < _SC_WORKERS - 1)
    def _():
        pltpu.sync_copy(tbl.at[pl.ds(0, _NPW)], out_hbm.at[pl.ds(lo, _NPW)])

    @pl.when(wid == _SC_WORKERS - 1)
    def _():
        last = 10000 - (_SC_WORKERS - 1) * _NPW
        pltpu.sync_copy(tbl.at[pl.ds(0, last)], out_hbm.at[pl.ds(lo, last)])


def _sc_scatmax_scan_body(r3_hbm, dix_hbm, out_hbm, eidl_hbm, dll_hbm, cnt_hbm,
                          dstbuf, mbuf, eidbuf, dlbuf, cbuf, tbl, gb, sem):
    wid = lax.axis_index("s") * _SC_CORES + lax.axis_index("c")
    lo = wid * _NPW
    hi = jnp.minimum(lo + _NPW, 10000)
    _zero_f32_2d(tbl, _NPW + 7, 4)
    zi = jnp.zeros((16,), jnp.int32)

    s319 = jnp.full((16,), 319, jnp.int32)

    def zb(i, _):
        eidbuf[pl.ds(i * 16, 16)] = zi
        dlbuf[pl.ds(i * 16, 16)] = s319
        return 0

    lax.fori_loop(0, _SCAP // 16, zb, 0)
    cbuf[pl.ds(0, 16)] = zi

    lane = lax.iota(jnp.int32, 16)

    def slice_body(s, cntv):
        pltpu.sync_copy(dix_hbm.at[s], dstbuf)

        def row_body(r, cntv):
            for g in range(_GCHUNK // 16):
                d = dstbuf[r, pl.ds(g * 16, 16)]
                m = (d >= lo) & (d < hi)
                base = s * 10000 + r * _GCHUNK + g * 16
                pos = cntv + plsc.cumsum(m.astype(jnp.int32)) - 1
                plsc.store_scatter(eidbuf, [pos], base + lane, mask=m)
                plsc.store_scatter(dlbuf, [pos], d - lo, mask=m)
                cntv = cntv + plsc.all_reduce_population_count(m)
            return cntv

        return lax.fori_loop(0, _GROWS_PW, row_body, cntv, unroll=2)

    cntv = lax.fori_loop(0, _SC_WORKERS, slice_body,
                         jnp.zeros((16,), jnp.int32))

    cbuf[pl.ds(0, 16)] = cntv
    cnt = cbuf[pl.ds(0, 16)][0]

    pltpu.sync_copy(eidbuf, eidl_hbm.at[wid])
    pltpu.sync_copy(dlbuf, dll_hbm.at[wid])
    pltpu.sync_copy(cbuf, cnt_hbm.at[wid])

    _apply_max(r3_hbm, eidbuf, dlbuf, tbl, gb, sem, cnt)
    _write_table(tbl, out_hbm, wid, lo)


def _sc_scatmax_reuse_body(r3_hbm, eidl_hbm, dll_hbm, cnt_hbm, out_hbm,
                           eidbuf, dlbuf, cbuf, tbl, gb, sem):
    wid = lax.axis_index("s") * _SC_CORES + lax.axis_index("c")
    lo = wid * _NPW
    _zero_f32_2d(tbl, _NPW + 7, 4)
    pltpu.sync_copy(eidl_hbm.at[wid], eidbuf)
    pltpu.sync_copy(dll_hbm.at[wid], dlbuf)
    pltpu.sync_copy(cnt_hbm.at[wid], cbuf)
    cnt = cbuf[pl.ds(0, 16)][0]
    _apply_max(r3_hbm, eidbuf, dlbuf, tbl, gb, sem, cnt)
    _write_table(tbl, out_hbm, wid, lo)


def _scat_scratch():
    return [
        pltpu.VMEM((_SCAP,), jnp.int32),
        pltpu.VMEM((_SCAP,), jnp.int32),
        pltpu.VMEM((16,), jnp.int32),
        pltpu.VMEM((_NPW + 7, 64), jnp.float32),
        pltpu.VMEM((2, _MCHUNK, 64), jnp.float32),
        pltpu.SemaphoreType.DMA,
    ]


def _sc_scatter_scan(r3, dix3d):
    mesh = plsc.VectorSubcoreMesh(core_axis_name="c", subcore_axis_name="s")
    fn = functools.partial(
        pl.kernel,
        mesh=mesh,
        compiler_params=pltpu.CompilerParams(use_tc_tiling_on_sc=False, needs_layout_passes=False),
        out_type=[
            jax.ShapeDtypeStruct((10000, 64), jnp.float32),
            jax.ShapeDtypeStruct((_SC_WORKERS, _SCAP), jnp.int32),
            jax.ShapeDtypeStruct((_SC_WORKERS, _SCAP), jnp.int32),
            jax.ShapeDtypeStruct((_SC_WORKERS, 16), jnp.int32),
        ],
        scratch_types=[pltpu.VMEM((_GROWS_PW, _GCHUNK), jnp.int32),
                       pltpu.VMEM((_SCAP,), jnp.int32)] + _scat_scratch(),
    )(_sc_scatmax_scan_body)
    return fn(r3, dix3d)


def _sc_scatter_reuse(r3, eidl, dll, cnts):
    mesh = plsc.VectorSubcoreMesh(core_axis_name="c", subcore_axis_name="s")
    fn = functools.partial(
        pl.kernel,
        mesh=mesh,
        compiler_params=pltpu.CompilerParams(use_tc_tiling_on_sc=False, needs_layout_passes=False),
        out_type=jax.ShapeDtypeStruct((10000, 64), jnp.float32),
        scratch_types=_scat_scratch(),
    )(_sc_scatmax_reuse_body)
    return fn(r3, eidl, dll, cnts)


def _precompute_body(pos_ref, feat_ref, wa_ref, wb_ref, wc_ref, b_ref, a_out, b_out):
    pos = pos_ref[...]
    feat = feat_ref[...]
    a = jnp.dot(pos, wa_ref[...], preferred_element_type=jnp.float32)
    a = a + jnp.dot(feat, wc_ref[...], preferred_element_type=jnp.float32)
    a_out[...] = a + b_ref[...]
    b_out[...] = jnp.dot(pos, wb_ref[...], preferred_element_type=jnp.float32)


def _node_precompute(pos, feat, w1, b1):
    n, f = feat.shape
    h = w1.shape[1]
    wa = w1[0:3] - w1[3:6]
    wb = w1[3:6]
    wc = w1[6:]
    grid = n // _NODE_BLK
    return pl.pallas_call(
        _precompute_body,
        grid=(grid,),
        in_specs=[
            pl.BlockSpec((_NODE_BLK, 3), lambda i: (i, 0)),
            pl.BlockSpec((_NODE_BLK, f), lambda i: (i, 0)),
            pl.BlockSpec((3, h), lambda i: (0, 0)),
            pl.BlockSpec((3, h), lambda i: (0, 0)),
            pl.BlockSpec((f, h), lambda i: (0, 0)),
            pl.BlockSpec((1, h), lambda i: (0, 0)),
        ],
        out_specs=[
            pl.BlockSpec((_NODE_BLK, h), lambda i: (i, 0)),
            pl.BlockSpec((_NODE_BLK, h), lambda i: (i, 0)),
        ],
        out_shape=[
            jax.ShapeDtypeStruct((n, h), jnp.float32),
            jax.ShapeDtypeStruct((n, h), jnp.float32),
        ],
    )(pos, feat, wa, wb, wc, b1.reshape(1, h))


def _edge_mlp_body(ga_ref, gb_ref, w2_ref, b2_ref, w3_ref, b3_ref, s3_ref, t3_ref, out_ref):
    h1 = jnp.maximum(ga_ref[...] + gb_ref[...], 0.0)
    u2 = jnp.dot(h1, w2_ref[...], preferred_element_type=jnp.float32) + b2_ref[...]
    h2 = jnp.maximum(u2, 0.0)
    u3 = jnp.dot(h2, w3_ref[...], preferred_element_type=jnp.float32) + b3_ref[...]
    h3 = s3_ref[...] * jnp.maximum(u3, 0.0) + t3_ref[...]
    out_ref[...] = jnp.maximum(h3, 0.0)


def _edge_mlp(ga, gb, w2f, b2f, w3f, b3f, s3, t3):
    e, h = ga.shape
    grid = e // _EDGE_BLK
    return pl.pallas_call(
        _edge_mlp_body,
        grid=(grid,),
        in_specs=[
            pl.BlockSpec((_EDGE_BLK, h), lambda i: (i, 0)),
            pl.BlockSpec((_EDGE_BLK, h), lambda i: (i, 0)),
            pl.BlockSpec((h, h), lambda i: (0, 0)),
            pl.BlockSpec((1, h), lambda i: (0, 0)),
            pl.BlockSpec((h, h), lambda i: (0, 0)),
            pl.BlockSpec((1, h), lambda i: (0, 0)),
            pl.BlockSpec((1, h), lambda i: (0, 0)),
            pl.BlockSpec((1, h), lambda i: (0, 0)),
        ],
        out_specs=pl.BlockSpec((_EDGE_BLK, h), lambda i: (i, 0)),
        out_shape=jax.ShapeDtypeStruct((e, h), jnp.float32),
    )(ga, gb, w2f, b2f.reshape(1, h), w3f, b3f.reshape(1, h),
      s3.reshape(1, h), t3.reshape(1, h))


def _head_body(m1_ref, m2_ref, lw1_ref, lb1_ref, lw2_ref, lb2_ref, out_ref):
    n = m1_ref.shape[0]
    mean1 = jnp.sum(m1_ref[...], axis=0, keepdims=True) * (1.0 / n)
    mean2 = jnp.sum(m2_ref[...], axis=0, keepdims=True) * (1.0 / n)
    j = jnp.concatenate([mean1, mean2], axis=1)
    z = jnp.maximum(jnp.dot(j, lw1_ref[...], preferred_element_type=jnp.float32)
                    + lb1_ref[...], 0.0)
    logits = jnp.dot(z, lw2_ref[...], preferred_element_type=jnp.float32) + lb2_ref[...]
    mx = jnp.max(logits, axis=1, keepdims=True)
    lse = jnp.log(jnp.sum(jnp.exp(logits - mx), axis=1, keepdims=True)) + mx
    out_ref[...] = logits - lse


def _head(m1, m2, lw1, lb1, lw2, lb2):
    n, h = m1.shape
    ncls = lw2.shape[1]
    return pl.pallas_call(
        _head_body,
        out_shape=jax.ShapeDtypeStruct((1, ncls), jnp.float32),
    )(m1, m2, lw1, lb1.reshape(1, h), lw2, lb2.reshape(1, ncls))


def _fold_bn(params):
    c = 1.0 / jnp.sqrt(1.0 + _BN_EPS)
    w1, b1, g1, be1 = params[0:4]
    w2, b2, g2, be2 = params[4:8]
    w3, b3, g3, be3 = params[8:12]
    s1, t1 = g1 * c, be1
    s2, t2 = g2 * c, be2
    s3, t3 = g3 * c, be3
    w2f = s1[:, None] * w2
    b2f = t1 @ w2 + b2
    w3f = s2[:, None] * w3
    b3f = t2 @ w3 + b3
    return w1, b1, w2f, b2f, w3f, b3f, s3, t3


def _conv_edge_mlp(pos, feat, dix2d, six2d, params):
    w1, b1, w2f, b2f, w3f, b3f, s3, t3 = _fold_bn(params)
    a, b = _node_precompute(pos, feat, w1, b1)
    ga, gb = _sc_gather(a, b, dix2d, six2d)
    return _edge_mlp(ga, gb, w2f, b2f, w3f, b3f, s3, t3)


def kernel(x, pos, edge_index, batch, p1, p2, lin):
    dst = edge_index[1]
    src = edge_index[0]
    dix2d = dst.reshape(_SC_WORKERS, _GROWS_PW, _GCHUNK)
    six2d = src.reshape(_SC_WORKERS, _GROWS_PW, _GCHUNK)
    r3a = _conv_edge_mlp(pos, x, dix2d, six2d, p1)
    h1, eidl, dll, cnts = _sc_scatter_scan(r3a, dix2d)
    r3b = _conv_edge_mlp(pos, h1, dix2d, six2d, p2)
    h2 = _sc_scatter_reuse(r3b, eidl, dll, cnts)
    lw1, lb1, lw2, lb2 = lin
    return _head(h1, h2, lw1, lb1, lw2, lb2)


# speedup vs baseline: 1.5590x; 1.0327x over previous
import functools
from functools import partial

import jax
import jax.numpy as jnp
from jax import lax
from jax.experimental import pallas as pl
from jax.experimental.pallas import tpu as pltpu
from jax.experimental.pallas import tpu_sc as plsc

_SC_CORES = 2
_SC_SUBCORES = 16
_SC_WORKERS = _SC_CORES * _SC_SUBCORES
_GCHUNK = 80
_GROWS = 4000
_GROWS_PW = _GROWS // _SC_WORKERS


def _sc_gather_body(a_hbm, b_hbm, dix_hbm, six_hbm, ga_hbm, gb_hbm,
                    dix_v, six_v, bufa, bufb, gsem, osem):
    wid = lax.axis_index("s") * _SC_CORES + lax.axis_index("c")
    row0 = wid * _GROWS_PW
    pltpu.sync_copy(dix_hbm.at[wid], dix_v)
    pltpu.sync_copy(six_hbm.at[wid], six_v)
    n = _GROWS_PW
    nb = 3

    def fire(k):
        slot = k % nb
        pltpu.async_copy(a_hbm.at[dix_v.at[k]], bufa.at[slot], gsem)
        pltpu.async_copy(b_hbm.at[six_v.at[k]], bufb.at[slot], gsem)

    def wait_in(k):
        slot = k % nb
        pltpu.make_async_copy(a_hbm.at[dix_v.at[k]], bufa.at[slot], gsem).wait()
        pltpu.make_async_copy(b_hbm.at[six_v.at[k]], bufb.at[slot], gsem).wait()

    def out_desc(k):
        slot = k % nb
        e0 = (row0 + k) * _GCHUNK
        da = (bufa.at[slot], ga_hbm.at[pl.ds(e0, _GCHUNK)])
        db = (bufb.at[slot], gb_hbm.at[pl.ds(e0, _GCHUNK)])
        return da, db

    fire(0)
    fire(1)

    def body(k, _):
        wait_in(k)
        (sa, dsta), (sb, dstb) = out_desc(k)
        pltpu.async_copy(sa, dsta, osem)
        pltpu.async_copy(sb, dstb, osem)

        @pl.when(k >= 1)
        def _():
            (pa, pda), (pb, pdb) = out_desc(k - 1)
            pltpu.make_async_copy(pa, pda, osem).wait()
            pltpu.make_async_copy(pb, pdb, osem).wait()

        @pl.when(k + 2 < n)
        def _():
            fire(k + 2)

        return 0

    lax.fori_loop(0, n, body, 0)
    (la, lda), (lb, ldb) = out_desc(n - 1)
    pltpu.make_async_copy(la, lda, osem).wait()
    pltpu.make_async_copy(lb, ldb, osem).wait()


def _sc_gather(a, b, dix2d, six2d):
    e = _GROWS * _GCHUNK
    h = a.shape[1]
    hb = b.shape[1]
    mesh = plsc.VectorSubcoreMesh(core_axis_name="c", subcore_axis_name="s")
    fn = functools.partial(
        pl.kernel,
        mesh=mesh,
        compiler_params=pltpu.CompilerParams(use_tc_tiling_on_sc=False, needs_layout_passes=False),
        out_type=[
            jax.ShapeDtypeStruct((e, h), jnp.float32),
            jax.ShapeDtypeStruct((e, hb), jnp.float32),
        ],
        scratch_types=[
            pltpu.VMEM((_GROWS_PW, _GCHUNK), jnp.int32),
            pltpu.VMEM((_GROWS_PW, _GCHUNK), jnp.int32),
            pltpu.VMEM((3, _GCHUNK, h), jnp.float32),
            pltpu.VMEM((3, _GCHUNK, hb), jnp.float32),
            pltpu.SemaphoreType.DMA,
            pltpu.SemaphoreType.DMA,
        ],
    )(_sc_gather_body)
    return fn(a, b, dix2d, six2d)

_BN_EPS = 1e-5
_N_NODES = 10000
_NODE_BLK = 1000
_EDGE_BLK = 2000


_NPW = 313
_SCAP = 16384
_MCHUNK = 80


def _zero_f32_2d(ref, nrows, ncols16):
    z = jnp.zeros((16,), jnp.float32)

    def body(r, _):
        for c in range(ncols16):
            ref[r, pl.ds(c * 16, 16)] = z
        return 0

    lax.fori_loop(0, nrows, body, 0)


def _apply_max(r3_hbm, eidbuf, dlbuf, tbl, gb, sem, cnt):
    nch = (cnt + _MCHUNK - 1) // _MCHUNK

    def fire(k, slot):
        return pltpu.async_copy(
            r3_hbm.at[eidbuf.at[pl.ds(k * _MCHUNK, _MCHUNK)]], gb.at[slot], sem)

    @pl.when(nch > 0)
    def _():
        fire(0, 0)

        def chunk(k, _):
            @pl.when(k + 1 < nch)
            def _():
                fire(k + 1, (k + 1) % 2)

            pltpu.make_async_copy(
                r3_hbm.at[eidbuf.at[pl.ds(k * _MCHUNK, _MCHUNK)]],
                gb.at[k % 2], sem).wait()
            slot = k % 2
            for b in range(_MCHUNK // 16):
                dls = dlbuf[pl.ds(k * _MCHUNK + b * 16, 16)]
                for j in range(16):
                    dl = dls[j]
                    i = b * 16 + j
                    for c in range(4):
                        sl = pl.ds(c * 16, 16)
                        tbl[dl, sl] = jnp.maximum(tbl[dl, sl], gb[slot, i, sl])
            return 0

        lax.fori_loop(0, nch, chunk, 0)


def _write_table(tbl, out_hbm, wid, lo):
    @pl.when(wid < _SC_WORKERS - 1)
    def _():
        pltpu.sync_copy(tbl.at[pl.ds(0, _NPW)], out_hbm.at[pl.ds(lo, _NPW)])

    @pl.when(wid == _SC_WORKERS - 1)
    def _():
        last = 10000 - (_SC_WORKERS - 1) * _NPW
        pltpu.sync_copy(tbl.at[pl.ds(0, last)], out_hbm.at[pl.ds(lo, last)])


def _sc_scatmax_scan_body(r3_hbm, dix_hbm, out_hbm, eidl_hbm, dll_hbm, cnt_hbm,
                          dstbuf, mbuf, eidbuf, dlbuf, cbuf, tbl, gb, sem):
    wid = lax.axis_index("s") * _SC_CORES + lax.axis_index("c")
    lo = wid * _NPW
    hi = jnp.minimum(lo + _NPW, 10000)
    _zero_f32_2d(tbl, _NPW + 7, 4)
    zi = jnp.zeros((16,), jnp.int32)

    s319 = jnp.full((16,), 319, jnp.int32)

    def zb(i, _):
        eidbuf[pl.ds(i * 16, 16)] = zi
        dlbuf[pl.ds(i * 16, 16)] = s319
        return 0

    lax.fori_loop(0, _SCAP // 16, zb, 0)
    cbuf[pl.ds(0, 16)] = zi

    lane = lax.iota(jnp.int32, 16)

    def slice_body(s, cntv):
        pltpu.sync_copy(dix_hbm.at[s], dstbuf)

        def row_body(r, cntv):
            for g in range(_GCHUNK // 16):
                d = dstbuf[r, pl.ds(g * 16, 16)]
                m = (d >= lo) & (d < hi)
                base = s * 10000 + r * _GCHUNK + g * 16
                pos = cntv + plsc.cumsum(m.astype(jnp.int32)) - 1
                plsc.store_scatter(eidbuf, [pos], base + lane, mask=m)
                plsc.store_scatter(dlbuf, [pos], d - lo, mask=m)
                cntv = cntv + plsc.all_reduce_population_count(m)
            return cntv

        return lax.fori_loop(0, _GROWS_PW, row_body, cntv, unroll=2)

    cntv = lax.fori_loop(0, _SC_WORKERS, slice_body,
                         jnp.zeros((16,), jnp.int32))

    cbuf[pl.ds(0, 16)] = cntv
    cnt = cbuf[pl.ds(0, 16)][0]

    pltpu.sync_copy(eidbuf, eidl_hbm.at[wid])
    pltpu.sync_copy(dlbuf, dll_hbm.at[wid])
    pltpu.sync_copy(cbuf, cnt_hbm.at[wid])

    _apply_max(r3_hbm, eidbuf, dlbuf, tbl, gb, sem, cnt)
    _write_table(tbl, out_hbm, wid, lo)


def _sc_scatmax_reuse_body(r3_hbm, eidl_hbm, dll_hbm, cnt_hbm, out_hbm,
                           eidbuf, dlbuf, cbuf, tbl, gb, sem):
    wid = lax.axis_index("s") * _SC_CORES + lax.axis_index("c")
    lo = wid * _NPW
    _zero_f32_2d(tbl, _NPW + 7, 4)
    pltpu.sync_copy(eidl_hbm.at[wid], eidbuf)
    pltpu.sync_copy(dll_hbm.at[wid], dlbuf)
    pltpu.sync_copy(cnt_hbm.at[wid], cbuf)
    cnt = cbuf[pl.ds(0, 16)][0]
    _apply_max(r3_hbm, eidbuf, dlbuf, tbl, gb, sem, cnt)
    _write_table(tbl, out_hbm, wid, lo)


def _scat_scratch():
    return [
        pltpu.VMEM((_SCAP,), jnp.int32),
        pltpu.VMEM((_SCAP,), jnp.int32),
        pltpu.VMEM((16,), jnp.int32),
        pltpu.VMEM((_NPW + 7, 64), jnp.float32),
        pltpu.VMEM((2, _MCHUNK, 64), jnp.float32),
        pltpu.SemaphoreType.DMA,
    ]


def _sc_scatter_scan(r3, dix3d):
    mesh = plsc.VectorSubcoreMesh(core_axis_name="c", subcore_axis_name="s")
    fn = functools.partial(
        pl.kernel,
        mesh=mesh,
        compiler_params=pltpu.CompilerParams(use_tc_tiling_on_sc=False, needs_layout_passes=False),
        out_type=[
            jax.ShapeDtypeStruct((10000, 64), jnp.float32),
            jax.ShapeDtypeStruct((_SC_WORKERS, _SCAP), jnp.int32),
            jax.ShapeDtypeStruct((_SC_WORKERS, _SCAP), jnp.int32),
            jax.ShapeDtypeStruct((_SC_WORKERS, 16), jnp.int32),
        ],
        scratch_types=[pltpu.VMEM((_GROWS_PW, _GCHUNK), jnp.int32),
                       pltpu.VMEM((_SCAP,), jnp.int32)] + _scat_scratch(),
    )(_sc_scatmax_scan_body)
    return fn(r3, dix3d)


def _sc_scatter_reuse(r3, eidl, dll, cnts):
    mesh = plsc.VectorSubcoreMesh(core_axis_name="c", subcore_axis_name="s")
    fn = functools.partial(
        pl.kernel,
        mesh=mesh,
        compiler_params=pltpu.CompilerParams(use_tc_tiling_on_sc=False, needs_layout_passes=False),
        out_type=jax.ShapeDtypeStruct((10000, 64), jnp.float32),
        scratch_types=_scat_scratch(),
    )(_sc_scatmax_reuse_body)
    return fn(r3, eidl, dll, cnts)


def _precompute_body(pos_ref, feat_ref, wa_ref, wb_ref, wc_ref, b_ref, a_out, b_out):
    pos = pos_ref[...]
    feat = feat_ref[...]
    a = jnp.dot(pos, wa_ref[...], preferred_element_type=jnp.float32)
    a = a + jnp.dot(feat, wc_ref[...], preferred_element_type=jnp.float32)
    a_out[...] = a + b_ref[...]
    b_out[...] = jnp.dot(pos, wb_ref[...], preferred_element_type=jnp.float32)


def _node_precompute(pos, feat, w1, b1):
    n, f = feat.shape
    h = w1.shape[1]
    wa = w1[0:3] - w1[3:6]
    wb = w1[3:6]
    wc = w1[6:]
    grid = n // _NODE_BLK
    return pl.pallas_call(
        _precompute_body,
        grid=(grid,),
        in_specs=[
            pl.BlockSpec((_NODE_BLK, 3), lambda i: (i, 0)),
            pl.BlockSpec((_NODE_BLK, f), lambda i: (i, 0)),
            pl.BlockSpec((3, h), lambda i: (0, 0)),
            pl.BlockSpec((3, h), lambda i: (0, 0)),
            pl.BlockSpec((f, h), lambda i: (0, 0)),
            pl.BlockSpec((1, h), lambda i: (0, 0)),
        ],
        out_specs=[
            pl.BlockSpec((_NODE_BLK, h), lambda i: (i, 0)),
            pl.BlockSpec((_NODE_BLK, h), lambda i: (i, 0)),
        ],
        out_shape=[
            jax.ShapeDtypeStruct((n, h), jnp.float32),
            jax.ShapeDtypeStruct((n, h), jnp.float32),
        ],
    )(pos, feat, wa, wb, wc, b1.reshape(1, h))


def _edge_mlp_body(ga_ref, gp_ref, w1b_ref, w2_ref, b2_ref, w3_ref, b3_ref, s3_ref, t3_ref, out_ref):
    u1 = ga_ref[...] + jnp.dot(gp_ref[...], w1b_ref[...],
                               preferred_element_type=jnp.float32)
    h1 = jnp.maximum(u1, 0.0)
    u2 = jnp.dot(h1, w2_ref[...], preferred_element_type=jnp.float32) + b2_ref[...]
    h2 = jnp.maximum(u2, 0.0)
    u3 = jnp.dot(h2, w3_ref[...], preferred_element_type=jnp.float32) + b3_ref[...]
    h3 = s3_ref[...] * jnp.maximum(u3, 0.0) + t3_ref[...]
    out_ref[...] = jnp.maximum(h3, 0.0)


def _edge_mlp(ga, gp, w1bp, w2f, b2f, w3f, b3f, s3, t3):
    e, h = ga.shape
    hb = gp.shape[1]
    grid = e // _EDGE_BLK
    return pl.pallas_call(
        _edge_mlp_body,
        grid=(grid,),
        in_specs=[
            pl.BlockSpec((_EDGE_BLK, h), lambda i: (i, 0)),
            pl.BlockSpec((_EDGE_BLK, hb), lambda i: (i, 0)),
            pl.BlockSpec((hb, h), lambda i: (0, 0)),
            pl.BlockSpec((h, h), lambda i: (0, 0)),
            pl.BlockSpec((1, h), lambda i: (0, 0)),
            pl.BlockSpec((h, h), lambda i: (0, 0)),
            pl.BlockSpec((1, h), lambda i: (0, 0)),
            pl.BlockSpec((1, h), lambda i: (0, 0)),
            pl.BlockSpec((1, h), lambda i: (0, 0)),
        ],
        out_specs=pl.BlockSpec((_EDGE_BLK, h), lambda i: (i, 0)),
        out_shape=jax.ShapeDtypeStruct((e, h), jnp.float32),
    )(ga, gp, w1bp, w2f, b2f.reshape(1, h), w3f, b3f.reshape(1, h),
      s3.reshape(1, h), t3.reshape(1, h))


def _head_body(m1_ref, m2_ref, lw1_ref, lb1_ref, lw2_ref, lb2_ref, out_ref):
    n = m1_ref.shape[0]
    mean1 = jnp.sum(m1_ref[...], axis=0, keepdims=True) * (1.0 / n)
    mean2 = jnp.sum(m2_ref[...], axis=0, keepdims=True) * (1.0 / n)
    j = jnp.concatenate([mean1, mean2], axis=1)
    z = jnp.maximum(jnp.dot(j, lw1_ref[...], preferred_element_type=jnp.float32)
                    + lb1_ref[...], 0.0)
    logits = jnp.dot(z, lw2_ref[...], preferred_element_type=jnp.float32) + lb2_ref[...]
    mx = jnp.max(logits, axis=1, keepdims=True)
    lse = jnp.log(jnp.sum(jnp.exp(logits - mx), axis=1, keepdims=True)) + mx
    out_ref[...] = logits - lse


def _head(m1, m2, lw1, lb1, lw2, lb2):
    n, h = m1.shape
    ncls = lw2.shape[1]
    return pl.pallas_call(
        _head_body,
        out_shape=jax.ShapeDtypeStruct((1, ncls), jnp.float32),
    )(m1, m2, lw1, lb1.reshape(1, h), lw2, lb2.reshape(1, ncls))


def _fold_bn(params):
    c = 1.0 / jnp.sqrt(1.0 + _BN_EPS)
    w1, b1, g1, be1 = params[0:4]
    w2, b2, g2, be2 = params[4:8]
    w3, b3, g3, be3 = params[8:12]
    s1, t1 = g1 * c, be1
    s2, t2 = g2 * c, be2
    s3, t3 = g3 * c, be3
    w2f = s1[:, None] * w2
    b2f = t1 @ w2 + b2
    w3f = s2[:, None] * w3
    b3f = t2 @ w3 + b3
    return w1, b1, w2f, b2f, w3f, b3f, s3, t3


def _conv_edge_mlp(pos, pos16, feat, dix2d, six2d, params):
    w1, b1, w2f, b2f, w3f, b3f, s3, t3 = _fold_bn(params)
    a, _ = _node_precompute(pos, feat, w1, b1)
    ga, gp = _sc_gather(a, pos16, dix2d, six2d)
    w1bp = jnp.concatenate(
        [w1[3:6], jnp.zeros((13, w1.shape[1]), jnp.float32)], axis=0)
    return _edge_mlp(ga, gp, w1bp, w2f, b2f, w3f, b3f, s3, t3)


def kernel(x, pos, edge_index, batch, p1, p2, lin):
    dst = edge_index[1]
    src = edge_index[0]
    dix2d = dst.reshape(_SC_WORKERS, _GROWS_PW, _GCHUNK)
    six2d = src.reshape(_SC_WORKERS, _GROWS_PW, _GCHUNK)
    pos16 = jnp.concatenate(
        [pos, jnp.zeros((pos.shape[0], 13), jnp.float32)], axis=1)
    r3a = _conv_edge_mlp(pos, pos16, x, dix2d, six2d, p1)
    h1, eidl, dll, cnts = _sc_scatter_scan(r3a, dix2d)
    r3b = _conv_edge_mlp(pos, pos16, h1, dix2d, six2d, p2)
    h2 = _sc_scatter_reuse(r3b, eidl, dll, cnts)
    lw1, lb1, lw2, lb2 = lin
    return _head(h1, h2, lw1, lb1, lw2, lb2)
